# Initial kernel scaffold; baseline (speedup 1.0000x reference)
import jax
import jax.numpy as jnp
from jax.experimental import pallas as pl


def kernel(residue, edge_index, weight):
    raise NotImplementedError("write your pallas kernel here")


# speedup vs baseline: 13.6867x; 13.6867x over previous
import jax
import jax.numpy as jnp
from jax import lax
from jax.experimental import pallas as pl
from jax.experimental.pallas import tpu as pltpu
from jax.experimental.pallas import tpu_sc as plsc

NUM_RESIDUE_TYPE = 32
PAIR_DIM = 128
N_NODES = 10000
N_EDGES = 320000

NC, NS, L = 2, 16, 16
NW = NC * NS
BPW = N_EDGES // NW
CHUNK = 80
NCHUNK = BPW // CHUNK
VECS = CHUNK // L


def _body(src_hbm, dst_hbm, residue_hbm, weight_hbm, out_hbm,
          src_v, dst_v, rs_v, rd_v, pair_c, rows_v, sem):
    wid = lax.axis_index("s") * NC + lax.axis_index("c")
    base = wid * BPW

    pltpu.sync_copy(src_hbm.at[pl.ds(base, BPW)], src_v)
    pltpu.sync_copy(dst_hbm.at[pl.ds(base, BPW)], dst_v)
    pltpu.async_copy(residue_hbm.at[src_v], rs_v, sem).wait()
    pltpu.async_copy(residue_hbm.at[dst_v], rd_v, sem).wait()

    def chunk_body(ci, carry):
        off = ci * CHUNK

        def vec_body(j, c):
            o = off + j * L
            rs = rs_v[pl.ds(o, L)]
            rd = rd_v[pl.ds(o, L)]
            pair_c[pl.ds(j * L, L)] = rs * NUM_RESIDUE_TYPE + rd
            return c

        lax.fori_loop(0, VECS, vec_body, 0, unroll=True)
        pltpu.async_copy(weight_hbm.at[pair_c], rows_v, sem).wait()
        pltpu.sync_copy(rows_v, out_hbm.at[pl.ds(base + off, CHUNK)])
        return carry

    lax.fori_loop(0, NCHUNK, chunk_body, 0)


@jax.jit
def kernel(residue, edge_index, weight):
    src = edge_index[:, 0].astype(jnp.int32)
    dst = edge_index[:, 1].astype(jnp.int32)
    mesh = plsc.VectorSubcoreMesh(core_axis_name="c", subcore_axis_name="s",
                                  num_cores=NC, num_subcores=NS)
    fn = pl.kernel(
        _body,
        out_type=jax.ShapeDtypeStruct((N_EDGES, PAIR_DIM), jnp.float32),
        mesh=mesh,
        scratch_types=[
            pltpu.VMEM((BPW,), jnp.int32),
            pltpu.VMEM((BPW,), jnp.int32),
            pltpu.VMEM((BPW,), jnp.int32),
            pltpu.VMEM((BPW,), jnp.int32),
            pltpu.VMEM((CHUNK,), jnp.int32),
            pltpu.VMEM((CHUNK, PAIR_DIM), jnp.float32),
            pltpu.SemaphoreType.DMA,
        ],
    )
    return fn(src, dst, residue.astype(jnp.int32), weight)


# speedup vs baseline: 17.2363x; 1.2594x over previous
import jax
import jax.numpy as jnp
from jax import lax
from jax.experimental import pallas as pl
from jax.experimental.pallas import tpu as pltpu
from jax.experimental.pallas import tpu_sc as plsc

NUM_RESIDUE_TYPE = 32
PAIR_DIM = 128
N_NODES = 10000
N_EDGES = 320000

NC, NS, L = 2, 16, 16
NW = NC * NS
BPW = N_EDGES // NW
CHUNK = 80
NCHUNK = BPW // CHUNK
VECS = CHUNK // L
K = 5
MACRO = NCHUNK // K


def _body(src_hbm, dst_hbm, residue_hbm, weight_hbm, out_hbm, *scratch):
    src_v, dst_v, rs_v, rd_v = scratch[0:4]
    pair_c = scratch[4:4 + K]
    rows = scratch[4 + K:4 + 2 * K]
    gsem = scratch[4 + 2 * K:4 + 3 * K]
    wsem = scratch[4 + 3 * K:4 + 4 * K]

    wid = lax.axis_index("s") * NC + lax.axis_index("c")
    base = wid * BPW

    pltpu.sync_copy(src_hbm.at[pl.ds(base, BPW)], src_v)
    pltpu.sync_copy(dst_hbm.at[pl.ds(base, BPW)], dst_v)
    pltpu.async_copy(residue_hbm.at[src_v], rs_v, gsem[0]).wait()
    pltpu.async_copy(residue_hbm.at[dst_v], rd_v, gsem[0]).wait()

    def macro_body(m, carry):
        for b in range(K):
            off = (m * K + b) * CHUNK
            for j in range(VECS):
                o = off + j * L
                pair_c[b][pl.ds(j * L, L)] = (
                    rs_v[pl.ds(o, L)] * NUM_RESIDUE_TYPE + rd_v[pl.ds(o, L)])

            @pl.when(m > 0)
            def _():
                pltpu.make_async_copy(
                    rows[b], out_hbm.at[pl.ds(base, CHUNK)], wsem[b]).wait()

            pltpu.async_copy(weight_hbm.at[pair_c[b]], rows[b], gsem[b])

        for b in range(K):
            off = (m * K + b) * CHUNK
            pltpu.make_async_copy(
                weight_hbm.at[pair_c[b]], rows[b], gsem[b]).wait()
            pltpu.async_copy(rows[b], out_hbm.at[pl.ds(base + off, CHUNK)],
                             wsem[b])
        return carry

    lax.fori_loop(0, MACRO, macro_body, 0)
    for b in range(K):
        pltpu.make_async_copy(
            rows[b], out_hbm.at[pl.ds(base, CHUNK)], wsem[b]).wait()


@jax.jit
def kernel(residue, edge_index, weight):
    src = edge_index[:, 0].astype(jnp.int32)
    dst = edge_index[:, 1].astype(jnp.int32)
    mesh = plsc.VectorSubcoreMesh(core_axis_name="c", subcore_axis_name="s",
                                  num_cores=NC, num_subcores=NS)
    scratch = (
        [pltpu.VMEM((BPW,), jnp.int32)] * 4
        + [pltpu.VMEM((CHUNK,), jnp.int32)] * K
        + [pltpu.VMEM((CHUNK, PAIR_DIM), jnp.float32)] * K
        + [pltpu.SemaphoreType.DMA] * (2 * K)
    )
    fn = pl.kernel(
        _body,
        out_type=jax.ShapeDtypeStruct((N_EDGES, PAIR_DIM), jnp.float32),
        mesh=mesh,
        scratch_types=scratch,
    )
    return fn(src, dst, residue.astype(jnp.int32), weight)


# speedup vs baseline: 18.9519x; 1.0995x over previous
import jax
import jax.numpy as jnp
from jax import lax
from jax.experimental import pallas as pl
from jax.experimental.pallas import tpu as pltpu
from jax.experimental.pallas import tpu_sc as plsc

NUM_RESIDUE_TYPE = 32
PAIR_DIM = 128
N_NODES = 10000
N_EDGES = 320000

NC, NS, L = 2, 16, 16
NW = NC * NS
BPW = N_EDGES // NW
CHUNK = 400
NCHUNK = BPW // CHUNK
VECS = CHUNK // L
K = 2
MACRO = NCHUNK // K
TAIL = NCHUNK - MACRO * K


def _body(src_hbm, dst_hbm, residue_hbm, weight_hbm, out_hbm, *scratch):
    sv = scratch[0:K]
    dv = scratch[K:2 * K]
    rs = scratch[2 * K:3 * K]
    rd = scratch[3 * K:4 * K]
    pair = scratch[4 * K:5 * K]
    rows = scratch[5 * K:6 * K]
    esem = scratch[6 * K:7 * K]
    rsem = scratch[7 * K:8 * K]
    gsem = scratch[8 * K:9 * K]
    wsem = scratch[9 * K:10 * K]

    wid = lax.axis_index("s") * NC + lax.axis_index("c")
    base = wid * BPW

    def stage_in(ci, b):
        off = base + ci * CHUNK
        pltpu.async_copy(src_hbm.at[pl.ds(off, CHUNK)], sv[b], esem[b])
        pltpu.async_copy(dst_hbm.at[pl.ds(off, CHUNK)], dv[b], esem[b])

    def fire_residue(b):
        pltpu.make_async_copy(src_hbm.at[pl.ds(0, CHUNK)], sv[b],
                              esem[b]).wait()
        pltpu.make_async_copy(dst_hbm.at[pl.ds(0, CHUNK)], dv[b],
                              esem[b]).wait()
        pltpu.async_copy(residue_hbm.at[sv[b]], rs[b], rsem[b])
        pltpu.async_copy(residue_hbm.at[dv[b]], rd[b], rsem[b])

    def fire_weight(b, reuse):
        pltpu.make_async_copy(residue_hbm.at[sv[b]], rs[b], rsem[b]).wait()
        pltpu.make_async_copy(residue_hbm.at[dv[b]], rd[b], rsem[b]).wait()

        def vec(j, c):
            o = j * L
            pair[b][pl.ds(o, L)] = (
                rs[b][pl.ds(o, L)] * NUM_RESIDUE_TYPE + rd[b][pl.ds(o, L)])
            return c

        lax.fori_loop(0, VECS, vec, 0, unroll=5)

        if reuse is not None:
            @pl.when(reuse)
            def _():
                pltpu.make_async_copy(
                    rows[b], out_hbm.at[pl.ds(base, CHUNK)], wsem[b]).wait()

        pltpu.async_copy(weight_hbm.at[pair[b]], rows[b], gsem[b])

    def fire_out(ci, b):
        pltpu.make_async_copy(weight_hbm.at[pair[b]], rows[b], gsem[b]).wait()
        pltpu.async_copy(rows[b], out_hbm.at[pl.ds(base + ci * CHUNK, CHUNK)],
                         wsem[b])

    def macro_body(m, carry):
        for b in range(K):
            stage_in(m * K + b, b)
        for b in range(K):
            fire_residue(b)
        for b in range(K):
            fire_weight(b, m > 0)
        for b in range(K):
            fire_out(m * K + b, b)
        return carry

    lax.fori_loop(0, MACRO, macro_body, 0)

    for t in range(TAIL):
        ci = MACRO * K + t
        stage_in(ci, t)
        fire_residue(t)
        fire_weight(t, jnp.bool_(True))
        fire_out(ci, t)

    for b in range(K):
        pltpu.make_async_copy(
            rows[b], out_hbm.at[pl.ds(base, CHUNK)], wsem[b]).wait()


@jax.jit
def kernel(residue, edge_index, weight):
    src = edge_index[:, 0].astype(jnp.int32)
    dst = edge_index[:, 1].astype(jnp.int32)
    mesh = plsc.VectorSubcoreMesh(core_axis_name="c", subcore_axis_name="s",
                                  num_cores=NC, num_subcores=NS)
    scratch = (
        [pltpu.VMEM((CHUNK,), jnp.int32) for _ in range(5 * K)]
        + [pltpu.VMEM((CHUNK, PAIR_DIM), jnp.float32) for _ in range(K)]
        + [pltpu.SemaphoreType.DMA for _ in range(4 * K)]
    )
    fn = pl.kernel(
        _body,
        out_type=jax.ShapeDtypeStruct((N_EDGES, PAIR_DIM), jnp.float32),
        mesh=mesh,
        scratch_types=scratch,
    )
    return fn(src, dst, residue.astype(jnp.int32), weight)


# speedup vs baseline: 35.8438x; 1.8913x over previous
import jax
import jax.numpy as jnp
from jax import lax
from jax.experimental import pallas as pl
from jax.experimental.pallas import tpu as pltpu
from jax.experimental.pallas import tpu_sc as plsc

NUM_RESIDUE_TYPE = 32
PAIR_DIM = 128
N_NODES = 10000
N_EDGES = 320000

NC, NS, L = 2, 16, 16
NW = NC * NS
BPW = N_EDGES // NW
CHUNK = 400
NCHUNK = BPW // CHUNK
VECS = CHUNK // L
K = 2
MACRO = NCHUNK // K
TAIL = NCHUNK - MACRO * K


def _body(src_hbm, dst_hbm, residue_hbm, weight_hbm, out_hbm, *scratch):
    sv = scratch[0:K]
    dv = scratch[K:2 * K]
    rs = scratch[2 * K:3 * K]
    rd = scratch[3 * K:4 * K]
    pair = scratch[4 * K:5 * K]
    rows = scratch[5 * K:6 * K]
    esem = scratch[6 * K:7 * K]
    rsem = scratch[7 * K:8 * K]
    gsem = scratch[8 * K:9 * K]
    wsem = scratch[9 * K:10 * K]

    shw = scratch[10 * K]
    shr = scratch[10 * K + 1]

    sid = lax.axis_index("s")
    wid = sid * NC + lax.axis_index("c")
    base = wid * BPW

    @pl.when(sid == 0)
    def _():
        pltpu.sync_copy(weight_hbm, shw)
        pltpu.sync_copy(residue_hbm, shr)

    plsc.subcore_barrier()

    def stage_in(ci, b):
        off = base + ci * CHUNK
        pltpu.async_copy(src_hbm.at[pl.ds(off, CHUNK)], sv[b], esem[b])
        pltpu.async_copy(dst_hbm.at[pl.ds(off, CHUNK)], dv[b], esem[b])

    def fire_residue(b):
        pltpu.make_async_copy(src_hbm.at[pl.ds(0, CHUNK)], sv[b],
                              esem[b]).wait()
        pltpu.make_async_copy(dst_hbm.at[pl.ds(0, CHUNK)], dv[b],
                              esem[b]).wait()
        pltpu.async_copy(shr.at[sv[b]], rs[b], rsem[b])
        pltpu.async_copy(shr.at[dv[b]], rd[b], rsem[b])

    def fire_weight(b, reuse):
        pltpu.make_async_copy(shr.at[sv[b]], rs[b], rsem[b]).wait()
        pltpu.make_async_copy(shr.at[dv[b]], rd[b], rsem[b]).wait()

        def vec(j, c):
            o = j * L
            pair[b][pl.ds(o, L)] = (
                rs[b][pl.ds(o, L)] * NUM_RESIDUE_TYPE + rd[b][pl.ds(o, L)])
            return c

        lax.fori_loop(0, VECS, vec, 0, unroll=5)

        if reuse is not None:
            @pl.when(reuse)
            def _():
                pltpu.make_async_copy(
                    rows[b], out_hbm.at[pl.ds(base, CHUNK)], wsem[b]).wait()

        pltpu.async_copy(shw.at[pair[b]], rows[b], gsem[b])

    def fire_out(ci, b):
        pltpu.make_async_copy(shw.at[pair[b]], rows[b], gsem[b]).wait()
        pltpu.async_copy(rows[b], out_hbm.at[pl.ds(base + ci * CHUNK, CHUNK)],
                         wsem[b])

    def macro_body(m, carry):
        for b in range(K):
            stage_in(m * K + b, b)
        for b in range(K):
            fire_residue(b)
        for b in range(K):
            fire_weight(b, m > 0)
        for b in range(K):
            fire_out(m * K + b, b)
        return carry

    lax.fori_loop(0, MACRO, macro_body, 0)

    for t in range(TAIL):
        ci = MACRO * K + t
        stage_in(ci, t)
        fire_residue(t)
        fire_weight(t, jnp.bool_(True))
        fire_out(ci, t)

    for b in range(K):
        pltpu.make_async_copy(
            rows[b], out_hbm.at[pl.ds(base, CHUNK)], wsem[b]).wait()


@jax.jit
def kernel(residue, edge_index, weight):
    src = edge_index[:, 0].astype(jnp.int32)
    dst = edge_index[:, 1].astype(jnp.int32)
    mesh = plsc.VectorSubcoreMesh(core_axis_name="c", subcore_axis_name="s",
                                  num_cores=NC, num_subcores=NS)
    scratch = (
        [pltpu.VMEM((CHUNK,), jnp.int32) for _ in range(5 * K)]
        + [pltpu.VMEM((CHUNK, PAIR_DIM), jnp.float32) for _ in range(K)]
        + [pltpu.SemaphoreType.DMA for _ in range(4 * K)]
        + [pltpu.VMEM_SHARED((NUM_RESIDUE_TYPE * NUM_RESIDUE_TYPE, PAIR_DIM),
                             jnp.float32),
           pltpu.VMEM_SHARED((N_NODES,), jnp.int32)]
    )
    fn = pl.kernel(
        _body,
        out_type=jax.ShapeDtypeStruct((N_EDGES, PAIR_DIM), jnp.float32),
        mesh=mesh,
        scratch_types=scratch,
    )
    return fn(src, dst, residue.astype(jnp.int32), weight)


# speedup vs baseline: 39.6661x; 1.1066x over previous
import jax
import jax.numpy as jnp
from jax import lax
from jax.experimental import pallas as pl
from jax.experimental.pallas import tpu as pltpu
from jax.experimental.pallas import tpu_sc as plsc

NUM_RESIDUE_TYPE = 32
PAIR_DIM = 128
N_NODES = 10000
N_EDGES = 320000

NC, NS, L = 2, 16, 16
NW = NC * NS
BPW = N_EDGES // NW
CHUNK = 80
NCHUNK = BPW // CHUNK
VECS = CHUNK // L
K = 6
MACRO = NCHUNK // K
TAIL = NCHUNK - MACRO * K


def _body(src_hbm, dst_hbm, residue_hbm, weight_hbm, out_hbm, *scratch):
    sv = scratch[0:K]
    dv = scratch[K:2 * K]
    rs = scratch[2 * K:3 * K]
    rd = scratch[3 * K:4 * K]
    pair = scratch[4 * K:5 * K]
    rows = scratch[5 * K:6 * K]
    esem = scratch[6 * K:7 * K]
    rsem = scratch[7 * K:8 * K]
    gsem = scratch[8 * K:9 * K]
    wsem = scratch[9 * K:10 * K]

    shw = scratch[10 * K]
    shr = scratch[10 * K + 1]

    sid = lax.axis_index("s")
    wid = sid * NC + lax.axis_index("c")
    base = wid * BPW

    @pl.when(sid == 0)
    def _():
        pltpu.sync_copy(weight_hbm, shw)
        pltpu.sync_copy(residue_hbm, shr)

    plsc.subcore_barrier()

    def stage_in(ci, b):
        off = base + ci * CHUNK
        pltpu.async_copy(src_hbm.at[pl.ds(off, CHUNK)], sv[b], esem[b])
        pltpu.async_copy(dst_hbm.at[pl.ds(off, CHUNK)], dv[b], esem[b])

    def fire_residue(b):
        pltpu.make_async_copy(src_hbm.at[pl.ds(0, CHUNK)], sv[b],
                              esem[b]).wait()
        pltpu.make_async_copy(dst_hbm.at[pl.ds(0, CHUNK)], dv[b],
                              esem[b]).wait()
        pltpu.async_copy(shr.at[sv[b]], rs[b], rsem[b])
        pltpu.async_copy(shr.at[dv[b]], rd[b], rsem[b])

    def fire_weight(b, reuse):
        pltpu.make_async_copy(shr.at[sv[b]], rs[b], rsem[b]).wait()
        pltpu.make_async_copy(shr.at[dv[b]], rd[b], rsem[b]).wait()

        def vec(j, c):
            o = j * L
            pair[b][pl.ds(o, L)] = (
                rs[b][pl.ds(o, L)] * NUM_RESIDUE_TYPE + rd[b][pl.ds(o, L)])
            return c

        lax.fori_loop(0, VECS, vec, 0, unroll=5)

        if reuse is not None:
            @pl.when(reuse)
            def _():
                pltpu.make_async_copy(
                    rows[b], out_hbm.at[pl.ds(base, CHUNK)], wsem[b]).wait()

        pltpu.async_copy(shw.at[pair[b]], rows[b], gsem[b])

    def fire_out(ci, b):
        pltpu.make_async_copy(shw.at[pair[b]], rows[b], gsem[b]).wait()
        pltpu.async_copy(rows[b], out_hbm.at[pl.ds(base + ci * CHUNK, CHUNK)],
                         wsem[b])

    def macro_body(m, carry):
        for b in range(K):
            stage_in(m * K + b, b)
        for b in range(K):
            fire_residue(b)
        for b in range(K):
            fire_weight(b, m > 0)
        for b in range(K):
            fire_out(m * K + b, b)
        return carry

    lax.fori_loop(0, MACRO, macro_body, 0)

    for t in range(TAIL):
        ci = MACRO * K + t
        stage_in(ci, t)
        fire_residue(t)
        fire_weight(t, jnp.bool_(True))
        fire_out(ci, t)

    for b in range(K):
        pltpu.make_async_copy(
            rows[b], out_hbm.at[pl.ds(base, CHUNK)], wsem[b]).wait()


@jax.jit
def kernel(residue, edge_index, weight):
    src = edge_index[:, 0].astype(jnp.int32)
    dst = edge_index[:, 1].astype(jnp.int32)
    mesh = plsc.VectorSubcoreMesh(core_axis_name="c", subcore_axis_name="s",
                                  num_cores=NC, num_subcores=NS)
    scratch = (
        [pltpu.VMEM((CHUNK,), jnp.int32) for _ in range(5 * K)]
        + [pltpu.VMEM((CHUNK, PAIR_DIM), jnp.float32) for _ in range(K)]
        + [pltpu.SemaphoreType.DMA for _ in range(4 * K)]
        + [pltpu.VMEM_SHARED((NUM_RESIDUE_TYPE * NUM_RESIDUE_TYPE, PAIR_DIM),
                             jnp.float32),
           pltpu.VMEM_SHARED((N_NODES,), jnp.int32)]
    )
    fn = pl.kernel(
        _body,
        out_type=jax.ShapeDtypeStruct((N_EDGES, PAIR_DIM), jnp.float32),
        mesh=mesh,
        scratch_types=scratch,
    )
    return fn(src, dst, residue.astype(jnp.int32), weight)


# speedup vs baseline: 40.6493x; 1.0248x over previous
import jax
import jax.numpy as jnp
from jax import lax
from jax.experimental import pallas as pl
from jax.experimental.pallas import tpu as pltpu
from jax.experimental.pallas import tpu_sc as plsc

NUM_RESIDUE_TYPE = 32
PAIR_DIM = 128
N_NODES = 10000
N_EDGES = 320000

NC, NS, L = 2, 16, 16
NW = NC * NS
BPW = N_EDGES // NW
CHUNK = 80
NCHUNK = BPW // CHUNK
VECS = CHUNK // L
K = 8
MACRO = NCHUNK // K
TAIL = NCHUNK - MACRO * K


def _body(src_hbm, dst_hbm, residue_hbm, weight_hbm, out_hbm, *scratch):
    sv = scratch[0:K]
    dv = scratch[K:2 * K]
    rs = scratch[2 * K:3 * K]
    rd = scratch[3 * K:4 * K]
    pair = scratch[4 * K:5 * K]
    rows = scratch[5 * K:6 * K]
    asem = scratch[6 * K:7 * K]
    wsem = scratch[7 * K:8 * K]

    shw = scratch[8 * K]
    shr = scratch[8 * K + 1]

    sid = lax.axis_index("s")
    wid = sid * NC + lax.axis_index("c")
    base = wid * BPW

    @pl.when(sid == 0)
    def _():
        pltpu.sync_copy(weight_hbm, shw)
        pltpu.sync_copy(residue_hbm, shr)

    plsc.subcore_barrier()

    def stage_in(ci, b):
        off = base + ci * CHUNK
        pltpu.async_copy(src_hbm.at[pl.ds(off, CHUNK)], sv[b], asem[b])
        pltpu.async_copy(dst_hbm.at[pl.ds(off, CHUNK)], dv[b], asem[b])

    def fire_residue(b):
        pltpu.make_async_copy(src_hbm.at[pl.ds(0, CHUNK)], sv[b],
                              asem[b]).wait()
        pltpu.make_async_copy(dst_hbm.at[pl.ds(0, CHUNK)], dv[b],
                              asem[b]).wait()
        pltpu.async_copy(shr.at[sv[b]], rs[b], asem[b])
        pltpu.async_copy(shr.at[dv[b]], rd[b], asem[b])

    def fire_weight(b, reuse):
        pltpu.make_async_copy(shr.at[sv[b]], rs[b], asem[b]).wait()
        pltpu.make_async_copy(shr.at[dv[b]], rd[b], asem[b]).wait()

        def vec(j, c):
            o = j * L
            pair[b][pl.ds(o, L)] = (
                rs[b][pl.ds(o, L)] * NUM_RESIDUE_TYPE + rd[b][pl.ds(o, L)])
            return c

        lax.fori_loop(0, VECS, vec, 0, unroll=5)

        if reuse is not None:
            @pl.when(reuse)
            def _():
                pltpu.make_async_copy(
                    rows[b], out_hbm.at[pl.ds(base, CHUNK)], wsem[b]).wait()

        pltpu.async_copy(shw.at[pair[b]], rows[b], asem[b])

    def fire_out(ci, b):
        pltpu.make_async_copy(shw.at[pair[b]], rows[b], asem[b]).wait()
        pltpu.async_copy(rows[b], out_hbm.at[pl.ds(base + ci * CHUNK, CHUNK)],
                         wsem[b])

    def macro_body(m, carry):
        for b in range(K):
            stage_in(m * K + b, b)
        for b in range(K):
            fire_residue(b)
        for b in range(K):
            fire_weight(b, m > 0)
        for b in range(K):
            fire_out(m * K + b, b)
        return carry

    lax.fori_loop(0, MACRO, macro_body, 0)

    for t in range(TAIL):
        ci = MACRO * K + t
        stage_in(ci, t)
        fire_residue(t)
        fire_weight(t, jnp.bool_(True))
        fire_out(ci, t)

    for b in range(K):
        pltpu.make_async_copy(
            rows[b], out_hbm.at[pl.ds(base, CHUNK)], wsem[b]).wait()


@jax.jit
def kernel(residue, edge_index, weight):
    src = edge_index[:, 0].astype(jnp.int32)
    dst = edge_index[:, 1].astype(jnp.int32)
    mesh = plsc.VectorSubcoreMesh(core_axis_name="c", subcore_axis_name="s",
                                  num_cores=NC, num_subcores=NS)
    scratch = (
        [pltpu.VMEM((CHUNK,), jnp.int32) for _ in range(5 * K)]
        + [pltpu.VMEM((CHUNK, PAIR_DIM), jnp.float32) for _ in range(K)]
        + [pltpu.SemaphoreType.DMA for _ in range(2 * K)]
        + [pltpu.VMEM_SHARED((NUM_RESIDUE_TYPE * NUM_RESIDUE_TYPE, PAIR_DIM),
                             jnp.float32),
           pltpu.VMEM_SHARED((N_NODES,), jnp.int32)]
    )
    fn = pl.kernel(
        _body,
        out_type=jax.ShapeDtypeStruct((N_EDGES, PAIR_DIM), jnp.float32),
        mesh=mesh,
        scratch_types=scratch,
    )
    return fn(src, dst, residue.astype(jnp.int32), weight)


# speedup vs baseline: 41.0850x; 1.0107x over previous
import jax
import jax.numpy as jnp
from jax import lax
from jax.experimental import pallas as pl
from jax.experimental.pallas import tpu as pltpu
from jax.experimental.pallas import tpu_sc as plsc

NUM_RESIDUE_TYPE = 32
PAIR_DIM = 128
N_NODES = 10000
N_EDGES = 320000

NC, NS, L = 2, 16, 16
NW = NC * NS
BPW = N_EDGES // NW
CHUNK = 80
NCHUNK = BPW // CHUNK
VECS = CHUNK // L
K = 11
MACRO = NCHUNK // K
TAIL = NCHUNK - MACRO * K


def _body(src_hbm, dst_hbm, residue_hbm, weight_hbm, out_hbm, *scratch):
    sv = scratch[0:K]
    dv = scratch[K:2 * K]
    rs = scratch[2 * K:3 * K]
    rd = scratch[3 * K:4 * K]
    pair = scratch[4 * K:5 * K]
    rows = scratch[5 * K:6 * K]
    asem = scratch[6 * K:7 * K]
    wsem = scratch[7 * K:8 * K]

    shw = scratch[8 * K]
    shr = scratch[8 * K + 1]

    sid = lax.axis_index("s")
    wid = sid * NC + lax.axis_index("c")
    base = wid * BPW

    @pl.when(sid == 0)
    def _():
        pltpu.sync_copy(weight_hbm, shw)
        pltpu.sync_copy(residue_hbm, shr)

    plsc.subcore_barrier()

    def stage_in(ci, b):
        off = base + ci * CHUNK
        pltpu.async_copy(src_hbm.at[pl.ds(off, CHUNK)], sv[b], asem[b])
        pltpu.async_copy(dst_hbm.at[pl.ds(off, CHUNK)], dv[b], asem[b])

    def fire_residue(b):
        pltpu.make_async_copy(src_hbm.at[pl.ds(0, CHUNK)], sv[b],
                              asem[b]).wait()
        pltpu.make_async_copy(dst_hbm.at[pl.ds(0, CHUNK)], dv[b],
                              asem[b]).wait()
        pltpu.async_copy(shr.at[sv[b]], rs[b], asem[b])
        pltpu.async_copy(shr.at[dv[b]], rd[b], asem[b])

    def fire_weight(b, reuse):
        pltpu.make_async_copy(shr.at[sv[b]], rs[b], asem[b]).wait()
        pltpu.make_async_copy(shr.at[dv[b]], rd[b], asem[b]).wait()

        def vec(j, c):
            o = j * L
            pair[b][pl.ds(o, L)] = (
                rs[b][pl.ds(o, L)] * NUM_RESIDUE_TYPE + rd[b][pl.ds(o, L)])
            return c

        lax.fori_loop(0, VECS, vec, 0, unroll=5)

        if reuse is not None:
            @pl.when(reuse)
            def _():
                pltpu.make_async_copy(
                    rows[b], out_hbm.at[pl.ds(base, CHUNK)], wsem[b]).wait()

        pltpu.async_copy(shw.at[pair[b]], rows[b], asem[b])

    def fire_out(ci, b):
        pltpu.make_async_copy(shw.at[pair[b]], rows[b], asem[b]).wait()
        pltpu.async_copy(rows[b], out_hbm.at[pl.ds(base + ci * CHUNK, CHUNK)],
                         wsem[b])

    def macro_body(m, carry):
        for b in range(K):
            stage_in(m * K + b, b)
        for b in range(K):
            fire_residue(b)
        for b in range(K):
            fire_weight(b, m > 0)
        for b in range(K):
            fire_out(m * K + b, b)
        return carry

    lax.fori_loop(0, MACRO, macro_body, 0)

    for t in range(TAIL):
        ci = MACRO * K + t
        stage_in(ci, t)
        fire_residue(t)
        fire_weight(t, jnp.bool_(True))
        fire_out(ci, t)

    for b in range(K):
        pltpu.make_async_copy(
            rows[b], out_hbm.at[pl.ds(base, CHUNK)], wsem[b]).wait()


@jax.jit
def kernel(residue, edge_index, weight):
    src = edge_index[:, 0].astype(jnp.int32)
    dst = edge_index[:, 1].astype(jnp.int32)
    mesh = plsc.VectorSubcoreMesh(core_axis_name="c", subcore_axis_name="s",
                                  num_cores=NC, num_subcores=NS)
    scratch = (
        [pltpu.VMEM((CHUNK,), jnp.int32) for _ in range(5 * K)]
        + [pltpu.VMEM((CHUNK, PAIR_DIM), jnp.float32) for _ in range(K)]
        + [pltpu.SemaphoreType.DMA for _ in range(2 * K)]
        + [pltpu.VMEM_SHARED((NUM_RESIDUE_TYPE * NUM_RESIDUE_TYPE, PAIR_DIM),
                             jnp.float32),
           pltpu.VMEM_SHARED((N_NODES,), jnp.int32)]
    )
    fn = pl.kernel(
        _body,
        out_type=jax.ShapeDtypeStruct((N_EDGES, PAIR_DIM), jnp.float32),
        mesh=mesh,
        scratch_types=scratch,
    )
    return fn(src, dst, residue.astype(jnp.int32), weight)
